# Initial kernel scaffold; baseline (speedup 1.0000x reference)
import jax
import jax.numpy as jnp
from jax.experimental import pallas as pl


def kernel(x, species, edge_src, edge_dst):
    raise NotImplementedError("write your pallas kernel here")


# speedup vs baseline: 5.2092x; 5.2092x over previous
import functools
import jax
import jax.numpy as jnp
from jax import lax
from jax.experimental import pallas as pl
from jax.experimental.pallas import tpu as pltpu
from jax.experimental.pallas import tpu_sc as plsc

_NC = 2
_NS = 16
_NW = _NC * _NS


def _gather_call(x, idx):
    E = idx.shape[0]
    D = x.shape[1]
    b_per_w = E // _NW
    C = 400
    n_chunks = b_per_w // C

    mesh = plsc.VectorSubcoreMesh(core_axis_name="c", subcore_axis_name="s")

    @functools.partial(
        pl.kernel,
        out_type=jax.ShapeDtypeStruct((E, D), jnp.float32),
        mesh=mesh,
        scratch_types=[
            pltpu.VMEM((C,), jnp.int32),
            pltpu.VMEM((C, D), jnp.float32),
            pltpu.SemaphoreType.DMA,
        ],
    )
    def gather_kernel(x_hbm, idx_hbm, out_hbm, idx_v, rows_v, sem):
        wid = lax.axis_index("s") * _NC + lax.axis_index("c")
        wbase = wid * b_per_w

        def body(i, carry):
            base = pl.multiple_of(wbase + i * C, 8)
            pltpu.sync_copy(idx_hbm.at[pl.ds(base, C)], idx_v)
            pltpu.async_copy(x_hbm.at[idx_v], rows_v, sem).wait()
            pltpu.sync_copy(rows_v, out_hbm.at[pl.ds(base, C)])
            return carry

        lax.fori_loop(0, n_chunks, body, 0)

    return gather_kernel(x, idx)


def kernel(x, species, edge_src, edge_dst):
    return _gather_call(x, edge_dst)


# speedup vs baseline: 5.9316x; 1.1387x over previous
import functools
import jax
import jax.numpy as jnp
from jax import lax
from jax.experimental import pallas as pl
from jax.experimental.pallas import tpu as pltpu
from jax.experimental.pallas import tpu_sc as plsc

_NC = 2
_NS = 16
_NW = _NC * _NS


def _gather_call(x, idx):
    E = idx.shape[0]
    D = x.shape[1]
    b_per_w = E // _NW
    C = 200
    n_chunks = b_per_w // C
    n_pairs = n_chunks // 2

    mesh = plsc.VectorSubcoreMesh(core_axis_name="c", subcore_axis_name="s")

    @functools.partial(
        pl.kernel,
        out_type=jax.ShapeDtypeStruct((E, D), jnp.float32),
        mesh=mesh,
        scratch_types=[
            pltpu.VMEM((C,), jnp.int32),
            pltpu.VMEM((C,), jnp.int32),
            pltpu.VMEM((C, D), jnp.float32),
            pltpu.VMEM((C, D), jnp.float32),
            pltpu.SemaphoreType.DMA,
            pltpu.SemaphoreType.DMA,
            pltpu.SemaphoreType.DMA,
            pltpu.SemaphoreType.DMA,
        ],
    )
    def gather_kernel(x_hbm, idx_hbm, out_hbm,
                      idx0, idx1, rows0, rows1, g0, g1, w0, w1):
        wid = lax.axis_index("s") * _NC + lax.axis_index("c")
        wbase = wid * b_per_w
        idx_b = (idx0, idx1)
        rows_b = (rows0, rows1)
        gsem = (g0, g1)
        wsem = (w0, w1)

        def chunk_base(c):
            return pl.multiple_of(wbase + c * C, 8)

        def issue_gather(c, b):
            pltpu.sync_copy(idx_hbm.at[pl.ds(chunk_base(c), C)], idx_b[b])
            pltpu.async_copy(x_hbm.at[idx_b[b]], rows_b[b], gsem[b])

        def wait_gather(b):
            pltpu.make_async_copy(
                x_hbm.at[idx_b[b]], rows_b[b], gsem[b]).wait()

        def issue_wb(c, b):
            pltpu.async_copy(
                rows_b[b], out_hbm.at[pl.ds(chunk_base(c), C)], wsem[b])

        def wait_wb(c, b):
            pltpu.make_async_copy(
                rows_b[b], out_hbm.at[pl.ds(chunk_base(c), C)],
                wsem[b]).wait()

        issue_gather(0, 0)
        issue_gather(1, 1)

        def body(j, carry):
            for b in range(2):
                c = 2 * j + b
                wait_gather(b)
                issue_wb(c, b)
                wait_wb(c, b)
                issue_gather(c + 2, b)
            return carry

        lax.fori_loop(0, n_pairs - 1, body, 0)

        for b in range(2):
            c = n_chunks - 2 + b
            wait_gather(b)
            issue_wb(c, b)
        for b in range(2):
            c = n_chunks - 2 + b
            wait_wb(c, b)

    return gather_kernel(x, idx)


def kernel(x, species, edge_src, edge_dst):
    return _gather_call(x, edge_dst)


# speedup vs baseline: 6.1746x; 1.0410x over previous
import functools
import jax
import jax.numpy as jnp
from jax import lax
from jax.experimental import pallas as pl
from jax.experimental.pallas import tpu as pltpu
from jax.experimental.pallas import tpu_sc as plsc

_NC = 2
_NS = 16
_NW = _NC * _NS
_C = 80
_NBUF = 5


def _gather_call(x, idx):
    E = idx.shape[0]
    D = x.shape[1]
    b_per_w = E // _NW
    C = _C
    nbuf = _NBUF
    n_chunks = b_per_w // C
    n_groups = n_chunks // nbuf

    mesh = plsc.VectorSubcoreMesh(core_axis_name="c", subcore_axis_name="s")

    @functools.partial(
        pl.kernel,
        out_type=jax.ShapeDtypeStruct((E, D), jnp.float32),
        mesh=mesh,
        scratch_types=(
            [pltpu.VMEM((b_per_w,), jnp.int32)]
            + [pltpu.VMEM((C, D), jnp.float32) for _ in range(nbuf)]
            + [pltpu.SemaphoreType.DMA for _ in range(2 * nbuf)]
        ),
    )
    def gather_kernel(x_hbm, idx_hbm, out_hbm, idx_v, *bufs):
        rows_b = bufs[:nbuf]
        gsem = bufs[nbuf:2 * nbuf]
        wsem = bufs[2 * nbuf:]
        wid = lax.axis_index("s") * _NC + lax.axis_index("c")
        wbase = wid * b_per_w

        pltpu.sync_copy(idx_hbm.at[pl.ds(pl.multiple_of(wbase, 8), b_per_w)],
                        idx_v)

        def issue_gather(c, b):
            lo = pl.multiple_of(c * C, 8)
            pltpu.async_copy(
                x_hbm.at[idx_v.at[pl.ds(lo, C)]], rows_b[b], gsem[b])

        def wait_gather(c, b):
            lo = pl.multiple_of(c * C, 8)
            pltpu.make_async_copy(
                x_hbm.at[idx_v.at[pl.ds(lo, C)]], rows_b[b], gsem[b]).wait()

        def issue_wb(c, b):
            lo = pl.multiple_of(wbase + c * C, 8)
            pltpu.async_copy(rows_b[b], out_hbm.at[pl.ds(lo, C)], wsem[b])

        def wait_wb(c, b):
            lo = pl.multiple_of(wbase + c * C, 8)
            pltpu.make_async_copy(
                rows_b[b], out_hbm.at[pl.ds(lo, C)], wsem[b]).wait()

        for b in range(nbuf):
            issue_gather(b, b)

        def body(j, carry):
            for b in range(nbuf):
                c = j * nbuf + b
                wait_gather(c, b)
                issue_wb(c, b)
                wait_wb(c, b)
                issue_gather(c + nbuf, b)
            return carry

        lax.fori_loop(0, n_groups - 1, body, 0)

        for b in range(nbuf):
            c = n_chunks - nbuf + b
            wait_gather(c, b)
            issue_wb(c, b)
        for b in range(nbuf):
            c = n_chunks - nbuf + b
            wait_wb(c, b)

    return gather_kernel(x, idx)


def kernel(x, species, edge_src, edge_dst):
    return _gather_call(x, edge_dst)
